# baseline (speedup 1.0000x reference)
import functools

import jax
import jax.numpy as jnp
from jax import lax
from jax.experimental import pallas as pl
from jax.experimental.pallas import tpu as pltpu
from jax.experimental.pallas import tpu_sc as plsc

_DIM = 768
_NE = 64
_NG = 8
_GS = _NE // _NG
_TOPK = 8
_TOPG = 4
_SCALE = 2.5
_L = 16


def _score_body(x_ref, w_ref, b_ref, out_ref, pk_ref):
    s = jax.lax.dot_general(
        w_ref[...], x_ref[...],
        dimension_numbers=(((1,), (1,)), ((), ())),
        preferred_element_type=jnp.float32,
    )
    scores = jax.nn.sigmoid(s) + b_ref[...]
    out_ref[...] = scores
    tb = scores.shape[1]
    neg = jnp.float32(-jnp.inf)
    gi = jax.lax.broadcasted_iota(jnp.int32, (_GS, tb), 0)
    gs_list = []
    for g in range(_NG):
        sub = scores[g * _GS:(g + 1) * _GS, :]
        m1 = jnp.max(sub, axis=0, keepdims=True)
        idx1 = jnp.min(jnp.where(sub == m1, gi, _GS), axis=0, keepdims=True)
        m2 = jnp.max(jnp.where(gi == idx1, neg, sub), axis=0, keepdims=True)
        gs_list.append(m1 + m2)
    gs = jnp.concatenate(gs_list, axis=0)
    gi8 = jax.lax.broadcasted_iota(jnp.int32, (_NG, tb), 0)
    packed = jnp.zeros((1, tb), dtype=jnp.int32)
    for r in range(_TOPG):
        gmax = jnp.max(gs, axis=0, keepdims=True)
        gidx = jnp.min(jnp.where(gs == gmax, gi8, _NG), axis=0, keepdims=True)
        packed = packed | (gidx << (3 * r))
        gs = jnp.where(gi8 == gidx, neg, gs)
    pk_ref[...] = packed


def _tree_max(vs):
    while len(vs) > 1:
        vs = [jnp.maximum(vs[i], vs[i + 1]) if i + 1 < len(vs) else vs[i]
              for i in range(0, len(vs), 2)]
    return vs[0]


def _tree_min(vs):
    while len(vs) > 1:
        vs = [jnp.minimum(vs[i], vs[i + 1]) if i + 1 < len(vs) else vs[i]
              for i in range(0, len(vs), 2)]
    return vs[0]


def _tree_add(vs):
    while len(vs) > 1:
        vs = [vs[i] + vs[i + 1] if i + 1 < len(vs) else vs[i]
              for i in range(0, len(vs), 2)]
    return vs[0]


def _make_sc_route(t, nc, ns):
    nw = nc * ns
    c = t // nw
    mesh = plsc.VectorSubcoreMesh(core_axis_name="c", subcore_axis_name="s")

    @functools.partial(
        pl.kernel, mesh=mesh,
        out_type=[
            jax.ShapeDtypeStruct((_TOPK, t), jnp.float32),
            jax.ShapeDtypeStruct((_TOPK, t), jnp.int32),
        ],
        scratch_types=[
            pltpu.VMEM((_NE, c), jnp.float32),
            pltpu.VMEM((1, c), jnp.int32),
            pltpu.VMEM((_TOPK, c), jnp.float32),
            pltpu.VMEM((_TOPK, c), jnp.int32),
        ],
    )
    def route(sc_hbm, pk_hbm, w_hbm, i_hbm, sv, pv, wv, iv):
        wid = lax.axis_index("s") * nc + lax.axis_index("c")
        base = wid * c
        pltpu.sync_copy(sc_hbm.at[:, pl.ds(base, c)], sv)
        pltpu.sync_copy(pk_hbm.at[:, pl.ds(base, c)], pv)

        neg = jnp.full((_L,), -jnp.inf, jnp.float32)
        seven = jnp.full((_L,), 7, jnp.int32)
        bige = jnp.full((_L,), _NE, jnp.int32)
        scale = jnp.full((_L,), _SCALE, jnp.float32)

        def block(j, carry):
            col0 = j * _L
            sl = pl.ds(col0, _L)
            v = [sv[e, sl] for e in range(_NE)]
            pk = pv[0, sl]
            selg = [(pk >> (3 * r)) & seven for r in range(_TOPG)]
            one = jnp.full((_L,), 1, jnp.int32)
            vals, ids = [], []
            for r in range(_TOPG):
                b0 = (selg[r] & one) == one
                b1 = ((selg[r] >> 1) & one) == one
                b2 = ((selg[r] >> 2) & one) == one
                gbase = selg[r] * _GS
                for q in range(_GS):
                    t0 = jnp.where(b0, v[1 * _GS + q], v[0 * _GS + q])
                    t1 = jnp.where(b0, v[3 * _GS + q], v[2 * _GS + q])
                    t2 = jnp.where(b0, v[5 * _GS + q], v[4 * _GS + q])
                    t3 = jnp.where(b0, v[7 * _GS + q], v[6 * _GS + q])
                    u0 = jnp.where(b1, t1, t0)
                    u1 = jnp.where(b1, t3, t2)
                    vals.append(jnp.where(b2, u1, u0))
                    ids.append(gbase + q)
            wouts, iouts = [], []
            for _ in range(_TOPK):
                m = _tree_max(vals)
                eidx = _tree_min(
                    [jnp.where(vals[s] == m, ids[s], bige)
                     for s in range(len(vals))])
                wouts.append(m)
                iouts.append(eidx)
                vals = [jnp.where(ids[s] == eidx, neg, vals[s])
                        for s in range(len(vals))]
            inv = scale / _tree_add(wouts)
            for k in range(_TOPK):
                wv[k, sl] = wouts[k] * inv
                iv[k, sl] = iouts[k]
            return carry

        lax.fori_loop(0, c // _L, block, jnp.int32(0))

        pltpu.sync_copy(wv, w_hbm.at[:, pl.ds(base, c)])
        pltpu.sync_copy(iv, i_hbm.at[:, pl.ds(base, c)])

    return route


def kernel(x, W, b):
    t = x.shape[0]
    tbm = 2048
    chunk_sizes = (12288, 12288, 4096, 4096)
    assert sum(chunk_sizes) == t
    b2 = b.reshape(_NE, 1)
    info = plsc.get_sparse_core_info()
    routes = {tc: _make_sc_route(tc, info.num_cores, info.num_subcores)
              for tc in set(chunk_sizes)}

    wts, its = [], []
    blk0 = 0
    for tc in chunk_sizes:
        nblk = tc // tbm
        scores_t, packed = pl.pallas_call(
            _score_body,
            grid=(nblk,),
            in_specs=[
                pl.BlockSpec((tbm, _DIM), lambda i, blk0=blk0: (blk0 + i, 0)),
                pl.BlockSpec((_NE, _DIM), lambda i: (0, 0)),
                pl.BlockSpec((_NE, 1), lambda i: (0, 0)),
            ],
            out_specs=[
                pl.BlockSpec((_NE, tbm), lambda i: (0, i)),
                pl.BlockSpec((1, tbm), lambda i: (0, i)),
            ],
            out_shape=[
                jax.ShapeDtypeStruct((_NE, tc), jnp.float32),
                jax.ShapeDtypeStruct((1, tc), jnp.int32),
            ],
        )(x, W, b2)
        wt, it = routes[tc](scores_t, packed)
        wts.append(wt)
        its.append(it)
        blk0 += nblk
    wt = jnp.concatenate(wts, axis=1)
    it = jnp.concatenate(its, axis=1)
    return wt.T, it.T


# speedup vs baseline: 1.1083x; 1.1083x over previous
import functools

import jax
import jax.numpy as jnp
from jax import lax
from jax.experimental import pallas as pl
from jax.experimental.pallas import tpu as pltpu
from jax.experimental.pallas import tpu_sc as plsc

_DIM = 768
_NE = 64
_NG = 8
_GS = _NE // _NG
_TOPK = 8
_TOPG = 4
_SCALE = 2.5
_L = 16


def _score_body(x_ref, w_ref, b_ref, out_ref, pk_ref):
    s = jax.lax.dot_general(
        w_ref[...], x_ref[...],
        dimension_numbers=(((1,), (1,)), ((), ())),
        preferred_element_type=jnp.float32,
    )
    scores = jax.nn.sigmoid(s) + b_ref[...]
    out_ref[...] = scores
    tb = scores.shape[1]
    neg = jnp.float32(-jnp.inf)
    gi = jax.lax.broadcasted_iota(jnp.int32, (_GS, tb), 0)
    gs_list = []
    for g in range(_NG):
        sub = scores[g * _GS:(g + 1) * _GS, :]
        m1 = jnp.max(sub, axis=0, keepdims=True)
        idx1 = jnp.min(jnp.where(sub == m1, gi, _GS), axis=0, keepdims=True)
        m2 = jnp.max(jnp.where(gi == idx1, neg, sub), axis=0, keepdims=True)
        gs_list.append(m1 + m2)
    gs = jnp.concatenate(gs_list, axis=0)
    gi8 = jax.lax.broadcasted_iota(jnp.int32, (_NG, tb), 0)
    packed = jnp.zeros((1, tb), dtype=jnp.int32)
    for r in range(_TOPG):
        gmax = jnp.max(gs, axis=0, keepdims=True)
        gidx = jnp.min(jnp.where(gs == gmax, gi8, _NG), axis=0, keepdims=True)
        packed = packed | (gidx << (3 * r))
        gs = jnp.where(gi8 == gidx, neg, gs)
    pk_ref[...] = packed


def _tree_max(vs):
    while len(vs) > 1:
        vs = [jnp.maximum(vs[i], vs[i + 1]) if i + 1 < len(vs) else vs[i]
              for i in range(0, len(vs), 2)]
    return vs[0]


def _tree_min(vs):
    while len(vs) > 1:
        vs = [jnp.minimum(vs[i], vs[i + 1]) if i + 1 < len(vs) else vs[i]
              for i in range(0, len(vs), 2)]
    return vs[0]


def _tree_add(vs):
    while len(vs) > 1:
        vs = [vs[i] + vs[i + 1] if i + 1 < len(vs) else vs[i]
              for i in range(0, len(vs), 2)]
    return vs[0]


def _make_sc_route(t, nc, ns):
    nw = nc * ns
    c = t // nw
    mesh = plsc.VectorSubcoreMesh(core_axis_name="c", subcore_axis_name="s")

    @functools.partial(
        pl.kernel, mesh=mesh,
        out_type=[
            jax.ShapeDtypeStruct((_TOPK, t), jnp.float32),
            jax.ShapeDtypeStruct((_TOPK, t), jnp.int32),
        ],
        scratch_types=[
            pltpu.VMEM((_NE, c), jnp.float32),
            pltpu.VMEM((1, c), jnp.int32),
            pltpu.VMEM((_TOPK, c), jnp.float32),
            pltpu.VMEM((_TOPK, c), jnp.int32),
        ],
    )
    def route(sc_hbm, pk_hbm, w_hbm, i_hbm, sv, pv, wv, iv):
        wid = lax.axis_index("s") * nc + lax.axis_index("c")
        base = wid * c
        pltpu.sync_copy(sc_hbm.at[:, pl.ds(base, c)], sv)
        pltpu.sync_copy(pk_hbm.at[:, pl.ds(base, c)], pv)

        neg = jnp.full((_L,), -jnp.inf, jnp.float32)
        seven = jnp.full((_L,), 7, jnp.int32)
        bige = jnp.full((_L,), _NE, jnp.int32)
        scale = jnp.full((_L,), _SCALE, jnp.float32)

        def block(j, carry):
            col0 = j * _L
            sl = pl.ds(col0, _L)
            v = [sv[e, sl] for e in range(_NE)]
            pk = pv[0, sl]
            selg = [(pk >> (3 * r)) & seven for r in range(_TOPG)]
            one = jnp.full((_L,), 1, jnp.int32)
            vals, ids = [], []
            for r in range(_TOPG):
                b0 = (selg[r] & one) == one
                b1 = ((selg[r] >> 1) & one) == one
                b2 = ((selg[r] >> 2) & one) == one
                gbase = selg[r] * _GS
                for q in range(_GS):
                    t0 = jnp.where(b0, v[1 * _GS + q], v[0 * _GS + q])
                    t1 = jnp.where(b0, v[3 * _GS + q], v[2 * _GS + q])
                    t2 = jnp.where(b0, v[5 * _GS + q], v[4 * _GS + q])
                    t3 = jnp.where(b0, v[7 * _GS + q], v[6 * _GS + q])
                    u0 = jnp.where(b1, t1, t0)
                    u1 = jnp.where(b1, t3, t2)
                    vals.append(jnp.where(b2, u1, u0))
                    ids.append(gbase + q)
            wouts, iouts = [], []
            for _ in range(_TOPK):
                m = _tree_max(vals)
                eidx = _tree_min(
                    [jnp.where(vals[s] == m, ids[s], bige)
                     for s in range(len(vals))])
                wouts.append(m)
                iouts.append(eidx)
                vals = [jnp.where(ids[s] == eidx, neg, vals[s])
                        for s in range(len(vals))]
            inv = scale / _tree_add(wouts)
            for k in range(_TOPK):
                wv[k, sl] = wouts[k] * inv
                iv[k, sl] = iouts[k]
            return carry

        lax.fori_loop(0, c // _L, block, jnp.int32(0))

        pltpu.sync_copy(wv, w_hbm.at[:, pl.ds(base, c)])
        pltpu.sync_copy(iv, i_hbm.at[:, pl.ds(base, c)])

    return route


def kernel(x, W, b):
    t = x.shape[0]
    tbm = 2048
    chunk_sizes = (16384, 16384)
    assert sum(chunk_sizes) == t
    b2 = b.reshape(_NE, 1)
    info = plsc.get_sparse_core_info()
    routes = {tc: _make_sc_route(tc, info.num_cores, info.num_subcores)
              for tc in set(chunk_sizes)}

    wts, its = [], []
    blk0 = 0
    for tc in chunk_sizes:
        nblk = tc // tbm
        scores_t, packed = pl.pallas_call(
            _score_body,
            grid=(nblk,),
            in_specs=[
                pl.BlockSpec((tbm, _DIM), lambda i, blk0=blk0: (blk0 + i, 0)),
                pl.BlockSpec((_NE, _DIM), lambda i: (0, 0)),
                pl.BlockSpec((_NE, 1), lambda i: (0, 0)),
            ],
            out_specs=[
                pl.BlockSpec((_NE, tbm), lambda i: (0, i)),
                pl.BlockSpec((1, tbm), lambda i: (0, i)),
            ],
            out_shape=[
                jax.ShapeDtypeStruct((_NE, tc), jnp.float32),
                jax.ShapeDtypeStruct((1, tc), jnp.int32),
            ],
        )(x, W, b2)
        wt, it = routes[tc](scores_t, packed)
        wts.append(wt)
        its.append(it)
        blk0 += nblk
    wt = jnp.concatenate(wts, axis=1)
    it = jnp.concatenate(its, axis=1)
    return wt.T, it.T


# speedup vs baseline: 1.1717x; 1.0572x over previous
import functools

import jax
import jax.numpy as jnp
from jax import lax
from jax.experimental import pallas as pl
from jax.experimental.pallas import tpu as pltpu
from jax.experimental.pallas import tpu_sc as plsc

_DIM = 768
_NE = 64
_NG = 8
_GS = _NE // _NG
_TOPK = 8
_TOPG = 4
_SCALE = 2.5
_L = 16


def _score_body(x_ref, w_ref, b_ref, out_ref, pk_ref):
    s = jax.lax.dot_general(
        w_ref[...], x_ref[...],
        dimension_numbers=(((1,), (1,)), ((), ())),
        preferred_element_type=jnp.float32,
    )
    scores = jax.nn.sigmoid(s) + b_ref[...]
    out_ref[...] = scores
    tb = scores.shape[1]
    neg = jnp.float32(-jnp.inf)
    gi = jax.lax.broadcasted_iota(jnp.int32, (_GS, tb), 0)
    gs_list = []
    for g in range(_NG):
        sub = scores[g * _GS:(g + 1) * _GS, :]
        m1 = jnp.max(sub, axis=0, keepdims=True)
        idx1 = jnp.min(jnp.where(sub == m1, gi, _GS), axis=0, keepdims=True)
        m2 = jnp.max(jnp.where(gi == idx1, neg, sub), axis=0, keepdims=True)
        gs_list.append(m1 + m2)
    gs = jnp.concatenate(gs_list, axis=0)
    gi8 = jax.lax.broadcasted_iota(jnp.int32, (_NG, tb), 0)
    packed = jnp.zeros((1, tb), dtype=jnp.int32)
    for r in range(_TOPG):
        gmax = jnp.max(gs, axis=0, keepdims=True)
        gidx = jnp.min(jnp.where(gs == gmax, gi8, _NG), axis=0, keepdims=True)
        packed = packed | (gidx << (3 * r))
        gs = jnp.where(gi8 == gidx, neg, gs)
    pk_ref[...] = packed


def _tree_max(vs):
    while len(vs) > 1:
        vs = [jnp.maximum(vs[i], vs[i + 1]) if i + 1 < len(vs) else vs[i]
              for i in range(0, len(vs), 2)]
    return vs[0]


def _tree_min(vs):
    while len(vs) > 1:
        vs = [jnp.minimum(vs[i], vs[i + 1]) if i + 1 < len(vs) else vs[i]
              for i in range(0, len(vs), 2)]
    return vs[0]


def _tree_add(vs):
    while len(vs) > 1:
        vs = [vs[i] + vs[i + 1] if i + 1 < len(vs) else vs[i]
              for i in range(0, len(vs), 2)]
    return vs[0]


def _make_sc_route(t, nc, ns):
    nw = nc * ns
    c = t // nw
    mesh = plsc.VectorSubcoreMesh(core_axis_name="c", subcore_axis_name="s")

    @functools.partial(
        pl.kernel, mesh=mesh,
        out_type=[
            jax.ShapeDtypeStruct((_TOPK, t), jnp.float32),
            jax.ShapeDtypeStruct((_TOPK, t), jnp.int32),
        ],
        scratch_types=[
            pltpu.VMEM((_NE, c), jnp.float32),
            pltpu.VMEM((1, c), jnp.int32),
            pltpu.VMEM((_TOPK, c), jnp.float32),
            pltpu.VMEM((_TOPK, c), jnp.int32),
        ],
    )
    def route(sc_hbm, pk_hbm, w_hbm, i_hbm, sv, pv, wv, iv):
        wid = lax.axis_index("s") * nc + lax.axis_index("c")
        base = wid * c
        pltpu.sync_copy(sc_hbm.at[:, pl.ds(base, c)], sv)
        pltpu.sync_copy(pk_hbm.at[:, pl.ds(base, c)], pv)

        neg = jnp.full((_L,), -jnp.inf, jnp.float32)
        seven = jnp.full((_L,), 7, jnp.int32)
        bige = jnp.full((_L,), _NE, jnp.int32)
        scale = jnp.full((_L,), _SCALE, jnp.float32)

        def block(j, carry):
            col0 = j * _L
            sl = pl.ds(col0, _L)
            v = [sv[e, sl] for e in range(_NE)]
            pk = pv[0, sl]
            selg = [(pk >> (3 * r)) & seven for r in range(_TOPG)]
            one = jnp.full((_L,), 1, jnp.int32)
            vals, ids = [], []
            for r in range(_TOPG):
                b0 = (selg[r] & one) == one
                b1 = ((selg[r] >> 1) & one) == one
                b2 = ((selg[r] >> 2) & one) == one
                gbase = selg[r] * _GS
                for q in range(_GS):
                    t0 = jnp.where(b0, v[1 * _GS + q], v[0 * _GS + q])
                    t1 = jnp.where(b0, v[3 * _GS + q], v[2 * _GS + q])
                    t2 = jnp.where(b0, v[5 * _GS + q], v[4 * _GS + q])
                    t3 = jnp.where(b0, v[7 * _GS + q], v[6 * _GS + q])
                    u0 = jnp.where(b1, t1, t0)
                    u1 = jnp.where(b1, t3, t2)
                    vals.append(jnp.where(b2, u1, u0))
                    ids.append(gbase + q)
            heads, hids, nexts, nids = [], [], [], []
            for p in range(len(vals) // 2):
                a, bb = vals[2 * p], vals[2 * p + 1]
                ia, ib = ids[2 * p], ids[2 * p + 1]
                cond = bb > a
                heads.append(jnp.maximum(a, bb))
                nexts.append(jnp.minimum(a, bb))
                hids.append(jnp.where(cond, ib, ia))
                nids.append(jnp.where(cond, ia, ib))
            np_ = len(heads)
            wouts, iouts = [], []
            for _ in range(_TOPK):
                m = _tree_max(heads)
                eidx = _tree_min(
                    [jnp.where(heads[p] == m, hids[p], bige)
                     for p in range(np_)])
                wouts.append(m)
                iouts.append(eidx)
                for p in range(np_):
                    win = hids[p] == eidx
                    heads[p] = jnp.where(win, nexts[p], heads[p])
                    hids[p] = jnp.where(win, nids[p], hids[p])
                    nexts[p] = jnp.where(win, neg, nexts[p])
            inv = scale / _tree_add(wouts)
            for k in range(_TOPK):
                wv[k, sl] = wouts[k] * inv
                iv[k, sl] = iouts[k]
            return carry

        lax.fori_loop(0, c // _L, block, jnp.int32(0))

        pltpu.sync_copy(wv, w_hbm.at[:, pl.ds(base, c)])
        pltpu.sync_copy(iv, i_hbm.at[:, pl.ds(base, c)])

    return route


def kernel(x, W, b):
    t = x.shape[0]
    tbm = 2048
    chunk_sizes = (16384, 16384)
    assert sum(chunk_sizes) == t
    b2 = b.reshape(_NE, 1)
    info = plsc.get_sparse_core_info()
    routes = {tc: _make_sc_route(tc, info.num_cores, info.num_subcores)
              for tc in set(chunk_sizes)}

    wts, its = [], []
    blk0 = 0
    for tc in chunk_sizes:
        nblk = tc // tbm
        scores_t, packed = pl.pallas_call(
            _score_body,
            grid=(nblk,),
            in_specs=[
                pl.BlockSpec((tbm, _DIM), lambda i, blk0=blk0: (blk0 + i, 0)),
                pl.BlockSpec((_NE, _DIM), lambda i: (0, 0)),
                pl.BlockSpec((_NE, 1), lambda i: (0, 0)),
            ],
            out_specs=[
                pl.BlockSpec((_NE, tbm), lambda i: (0, i)),
                pl.BlockSpec((1, tbm), lambda i: (0, i)),
            ],
            out_shape=[
                jax.ShapeDtypeStruct((_NE, tc), jnp.float32),
                jax.ShapeDtypeStruct((1, tc), jnp.int32),
            ],
        )(x, W, b2)
        wt, it = routes[tc](scores_t, packed)
        wts.append(wt)
        its.append(it)
        blk0 += nblk
    wt = jnp.concatenate(wts, axis=1)
    it = jnp.concatenate(its, axis=1)
    return wt.T, it.T
